# Initial kernel scaffold; baseline (speedup 1.0000x reference)
import functools

import jax
import jax.numpy as jnp
from jax import lax
from jax.experimental import pallas as pl
from jax.experimental.pallas import tpu as pltpu
from jax.experimental.pallas import tpu_sc as plsc

D_MODEL = 64
CHUNK = 128

_info = plsc.get_sparse_core_info()
_NC, _NS = _info.num_cores, _info.num_subcores
NW = _NC * _NS


def _make_sc_lookup(n_chunks: int):
    mesh = plsc.VectorSubcoreMesh(core_axis_name="c", subcore_axis_name="s")

    @functools.partial(
        pl.kernel,
        mesh=mesh,
        out_type=jax.ShapeDtypeStruct((NW, n_chunks, CHUNK, D_MODEL), jnp.float32),
        scratch_types=[
            pltpu.VMEM((n_chunks, CHUNK), jnp.int32),
            pltpu.VMEM((CHUNK, D_MODEL), jnp.float32),
            pltpu.SemaphoreType.DMA,
        ],
    )
    def sc_lookup(x_hbm, table_hbm, out_hbm, idx_v, rows_v, sem):
        wid = lax.axis_index("s") * _NC + lax.axis_index("c")
        pltpu.sync_copy(x_hbm.at[wid], idx_v)

        def step(j, carry):
            pltpu.async_copy(table_hbm.at[idx_v.at[j]], rows_v, sem).wait()
            pltpu.sync_copy(rows_v, out_hbm.at[wid, j])
            return carry

        lax.fori_loop(0, n_chunks, step, 0)

    return sc_lookup


def kernel(x, emb_table):
    batch, seq = x.shape
    total = batch * seq
    assert total % (NW * CHUNK) == 0
    n_chunks = total // (NW * CHUNK)
    x32 = x.astype(jnp.int32).reshape(NW, n_chunks, CHUNK)
    out = _make_sc_lookup(n_chunks)(x32, emb_table)
    return out.reshape(batch, seq, D_MODEL)


# speedup vs baseline: 3.9584x; 3.9584x over previous
import functools

import jax
import jax.numpy as jnp
from jax import lax
from jax.experimental import pallas as pl
from jax.experimental.pallas import tpu as pltpu
from jax.experimental.pallas import tpu_sc as plsc

D_MODEL = 64
CHUNK = 128

_info = plsc.get_sparse_core_info()
_NC, _NS = _info.num_cores, _info.num_subcores
NW = _NC * _NS


def _make_sc_lookup(n_chunks: int):
    mesh = plsc.VectorSubcoreMesh(core_axis_name="c", subcore_axis_name="s")

    @functools.partial(
        pl.kernel,
        mesh=mesh,
        out_type=jax.ShapeDtypeStruct((NW, n_chunks, CHUNK, D_MODEL), jnp.float32),
        scratch_types=[
            pltpu.VMEM((n_chunks, CHUNK), jnp.int32),
            pltpu.VMEM((CHUNK, D_MODEL), jnp.float32),
            pltpu.SemaphoreType.DMA,
        ],
        compiler_params=pltpu.CompilerParams(use_tc_tiling_on_sc=False),
    )
    def sc_lookup(x_hbm, table_hbm, out_hbm, idx_v, rows_v, sem):
        wid = lax.axis_index("s") * _NC + lax.axis_index("c")
        pltpu.sync_copy(x_hbm.at[wid], idx_v)

        def step(j, carry):
            pltpu.async_copy(table_hbm.at[idx_v.at[j]], rows_v, sem).wait()
            pltpu.sync_copy(rows_v, out_hbm.at[wid, j])
            return carry

        lax.fori_loop(0, n_chunks, step, 0)

    return sc_lookup


def kernel(x, emb_table):
    batch, seq = x.shape
    total = batch * seq
    assert total % (NW * CHUNK) == 0
    n_chunks = total // (NW * CHUNK)
    x32 = x.astype(jnp.int32).reshape(NW, n_chunks, CHUNK)
    out = _make_sc_lookup(n_chunks)(x32, emb_table)
    return out.reshape(batch, seq, D_MODEL)


# speedup vs baseline: 4.1202x; 1.0409x over previous
import functools

import jax
import jax.numpy as jnp
from jax import lax
from jax.experimental import pallas as pl
from jax.experimental.pallas import tpu as pltpu
from jax.experimental.pallas import tpu_sc as plsc

D_MODEL = 64
CHUNK = 128
K = 4

_info = plsc.get_sparse_core_info()
_NC, _NS = _info.num_cores, _info.num_subcores
NW = _NC * _NS


def _make_sc_lookup(n_groups: int):
    mesh = plsc.VectorSubcoreMesh(core_axis_name="c", subcore_axis_name="s")

    @functools.partial(
        pl.kernel,
        mesh=mesh,
        out_type=jax.ShapeDtypeStruct(
            (NW, n_groups, K, CHUNK, D_MODEL), jnp.float32
        ),
        scratch_types=[
            pltpu.VMEM((K, CHUNK), jnp.int32),
            pltpu.VMEM((K, CHUNK), jnp.int32),
            pltpu.VMEM((K, CHUNK, D_MODEL), jnp.float32),
            pltpu.VMEM((K, CHUNK, D_MODEL), jnp.float32),
            pltpu.SemaphoreType.DMA,
            pltpu.SemaphoreType.DMA,
        ],
        compiler_params=pltpu.CompilerParams(use_tc_tiling_on_sc=False),
    )
    def sc_lookup(x_hbm, table_hbm, out_hbm, idx_a, idx_b, rows_a, rows_b,
                  gsem_a, gsem_b):
        wid = lax.axis_index("s") * _NC + lax.axis_index("c")

        def fire(idx_v, rows_v, gsem, g):
            pltpu.sync_copy(x_hbm.at[wid, g], idx_v)
            for b in range(K):
                pltpu.async_copy(table_hbm.at[idx_v.at[b]], rows_v.at[b], gsem)

        def phase(idx_v, rows_v, gsem, g):
            for b in range(K):
                pltpu.make_async_copy(
                    table_hbm.at[idx_v.at[b]], rows_v.at[b], gsem
                ).wait()
            pltpu.sync_copy(rows_v, out_hbm.at[wid, g])

            @pl.when(g + 2 < n_groups)
            def _():
                fire(idx_v, rows_v, gsem, g + 2)

        fire(idx_a, rows_a, gsem_a, 0)
        fire(idx_b, rows_b, gsem_b, 1)

        def body(p, carry):
            phase(idx_a, rows_a, gsem_a, 2 * p)
            phase(idx_b, rows_b, gsem_b, 2 * p + 1)
            return carry

        lax.fori_loop(0, n_groups // 2, body, 0)

    return sc_lookup


def kernel(x, emb_table):
    batch, seq = x.shape
    total = batch * seq
    assert total % (NW * K * CHUNK * 2) == 0
    n_groups = total // (NW * K * CHUNK)
    x32 = x.astype(jnp.int32).reshape(NW, n_groups, K, CHUNK)
    out = _make_sc_lookup(n_groups)(x32, emb_table)
    return out.reshape(batch, seq, D_MODEL)


# speedup vs baseline: 4.1279x; 1.0019x over previous
import functools

import jax
import jax.numpy as jnp
from jax import lax
from jax.experimental import pallas as pl
from jax.experimental.pallas import tpu as pltpu
from jax.experimental.pallas import tpu_sc as plsc

D_MODEL = 64
CHUNK = 128
K = 4

_info = plsc.get_sparse_core_info()
_NC, _NS = _info.num_cores, _info.num_subcores
NW = _NC * _NS


def _make_sc_lookup(n_groups: int):
    mesh = plsc.VectorSubcoreMesh(core_axis_name="c", subcore_axis_name="s")

    @functools.partial(
        pl.kernel,
        mesh=mesh,
        out_type=jax.ShapeDtypeStruct(
            (NW * n_groups * K * CHUNK, D_MODEL), jnp.float32
        ),
        scratch_types=[
            pltpu.VMEM((K, CHUNK), jnp.int32),
            pltpu.VMEM((K, CHUNK), jnp.int32),
            pltpu.VMEM((K * CHUNK, D_MODEL), jnp.float32),
            pltpu.VMEM((K * CHUNK, D_MODEL), jnp.float32),
            pltpu.SemaphoreType.DMA,
            pltpu.SemaphoreType.DMA,
        ],
        compiler_params=pltpu.CompilerParams(use_tc_tiling_on_sc=False),
    )
    def sc_lookup(x_hbm, table_hbm, out_hbm, idx_a, idx_b, rows_a, rows_b,
                  gsem_a, gsem_b):
        wid = lax.axis_index("s") * _NC + lax.axis_index("c")
        woff = wid * (n_groups * K * CHUNK)

        def fire(idx_v, rows_v, gsem, g):
            pltpu.sync_copy(x_hbm.at[wid, g], idx_v)
            for b in range(K):
                pltpu.async_copy(
                    table_hbm.at[idx_v.at[b]],
                    rows_v.at[pl.ds(b * CHUNK, CHUNK)],
                    gsem,
                )

        def phase(idx_v, rows_v, gsem, g):
            for b in range(K):
                pltpu.make_async_copy(
                    table_hbm.at[idx_v.at[b]],
                    rows_v.at[pl.ds(b * CHUNK, CHUNK)],
                    gsem,
                ).wait()
            pltpu.sync_copy(
                rows_v, out_hbm.at[pl.ds(woff + g * (K * CHUNK), K * CHUNK)]
            )

            @pl.when(g + 2 < n_groups)
            def _():
                fire(idx_v, rows_v, gsem, g + 2)

        fire(idx_a, rows_a, gsem_a, 0)
        fire(idx_b, rows_b, gsem_b, 1)

        def body(p, carry):
            phase(idx_a, rows_a, gsem_a, 2 * p)
            phase(idx_b, rows_b, gsem_b, 2 * p + 1)
            return carry

        lax.fori_loop(0, n_groups // 2, body, 0)

    return sc_lookup


def kernel(x, emb_table):
    batch, seq = x.shape
    total = batch * seq
    assert total % (NW * K * CHUNK * 2) == 0
    n_groups = total // (NW * K * CHUNK)
    x32 = x.astype(jnp.int32).reshape(NW, n_groups, K, CHUNK)
    out = _make_sc_lookup(n_groups)(x32, emb_table)
    return out.reshape(batch, seq, D_MODEL)
